# baseline (speedup 1.0000x reference)
import jax
import jax.numpy as jnp
from jax.experimental import pallas as pl
from jax.experimental.pallas import tpu as pltpu

BATCH = 1024
EMBED = 768
PARTITION = 96
CENTROIDS = 256
DSUB = 8
GROUP = 16
NGROUPS = PARTITION // GROUP
SEG = GROUP * CENTROIDS


SUBG = 4
NSUB = GROUP // SUBG
SUBW = SUBG * DSUB
SUBCOLS = SUBG * CENTROIDS


def _quant_group(vec_ref, cbt_ref, cb_ref, out_ref, w_ref, c_ref):
    w_ref[...] = jnp.zeros((GROUP * DSUB, SUBCOLS), jnp.float32)
    c_ref[...] = jnp.zeros((SEG, GROUP * DSUB), jnp.bfloat16)
    for q in range(GROUP):
        h, j = divmod(q, SUBG)
        w_ref[h * SUBW + j * DSUB:h * SUBW + (j + 1) * DSUB,
              j * CENTROIDS:(j + 1) * CENTROIDS] = cbt_ref[q]
        c_ref[q * CENTROIDS:(q + 1) * CENTROIDS, q * DSUB:(q + 1) * DSUB] = (
            cb_ref[q].astype(jnp.bfloat16))

    v = vec_ref[...]
    iota = jax.lax.broadcasted_iota(
        jnp.int32, (BATCH, CENTROIDS), 1).astype(jnp.float32)

    for h in range(NSUB):
        w_h = w_ref[h * SUBW:(h + 1) * SUBW, :]
        cnorm = jnp.sum(w_h * w_h, axis=0, keepdims=True)
        v_h = v[:, h * SUBW:(h + 1) * SUBW]
        scores = jax.lax.dot_general(
            v_h, w_h, (((1,), (0,)), ((), ())),
            precision=jax.lax.Precision.HIGHEST,
            preferred_element_type=jnp.float32)
        adj = 2.0 * scores - cnorm

        hots = []
        for j in range(SUBG):
            seg = adj[:, j * CENTROIDS:(j + 1) * CENTROIDS]
            m = jnp.max(seg, axis=1, keepdims=True)
            cand = jnp.where(seg == m, iota, float(CENTROIDS))
            idx = jnp.min(cand, axis=1, keepdims=True)
            hots.append((iota == idx).astype(jnp.bfloat16))
        hot = jnp.concatenate(hots, axis=1)
        c_h = c_ref[h * SUBCOLS:(h + 1) * SUBCOLS,
                    h * SUBW:(h + 1) * SUBW]
        out_ref[:, h * SUBW:(h + 1) * SUBW] = jax.lax.dot_general(
            hot, c_h, (((1,), (0,)), ((), ())),
            preferred_element_type=jnp.float32)


@jax.jit
def kernel(vecs, codebook):
    cbt = codebook.transpose(0, 2, 1)
    return pl.pallas_call(
        _quant_group,
        grid=(NGROUPS,),
        in_specs=[
            pl.BlockSpec((BATCH, GROUP * DSUB), lambda g: (0, g)),
            pl.BlockSpec((GROUP, DSUB, CENTROIDS), lambda g: (g, 0, 0)),
            pl.BlockSpec((GROUP, CENTROIDS, DSUB), lambda g: (g, 0, 0)),
        ],
        out_specs=pl.BlockSpec((BATCH, GROUP * DSUB), lambda g: (0, g)),
        out_shape=jax.ShapeDtypeStruct((BATCH, EMBED), jnp.float32),
        scratch_shapes=[
            pltpu.VMEM((GROUP * DSUB, SUBCOLS), jnp.float32),
            pltpu.VMEM((SEG, GROUP * DSUB), jnp.bfloat16),
        ],
    )(vecs, cbt, codebook)


# speedup vs baseline: 1.1686x; 1.1686x over previous
import jax
import jax.numpy as jnp
from jax.experimental import pallas as pl
from jax.experimental.pallas import tpu as pltpu

BATCH = 1024
EMBED = 768
PARTITION = 96
CENTROIDS = 256
DSUB = 8
GROUP = 16
NGROUPS = PARTITION // GROUP
SEG = GROUP * CENTROIDS


def _quant_group(vec_ref, cbt_ref, cb_ref, out_ref, w_ref, c_ref):
    w_ref[...] = jnp.zeros((GROUP * DSUB, SEG), jnp.float32)
    c_ref[...] = jnp.zeros((SEG, GROUP * DSUB), jnp.bfloat16)
    for q in range(GROUP):
        w_ref[q * DSUB:(q + 1) * DSUB, q * CENTROIDS:(q + 1) * CENTROIDS] = cbt_ref[q]
        c_ref[q * CENTROIDS:(q + 1) * CENTROIDS, q * DSUB:(q + 1) * DSUB] = (
            cb_ref[q].astype(jnp.bfloat16))

    v = vec_ref[...]
    iota = jax.lax.broadcasted_iota(
        jnp.int32, (BATCH, CENTROIDS), 1).astype(jnp.float32)

    NSPLIT = 2
    NCOLS = SEG // NSPLIT
    hots = []
    for h in range(NSPLIT):
        w_h = w_ref[:, h * NCOLS:(h + 1) * NCOLS]
        cnorm = jnp.sum(w_h * w_h, axis=0, keepdims=True)
        scores = jax.lax.dot_general(
            v, w_h, (((1,), (0,)), ((), ())),
            precision=jax.lax.Precision.HIGHEST,
            preferred_element_type=jnp.float32)
        adj = 2.0 * scores - cnorm

        for q in range(NCOLS // CENTROIDS):
            seg = adj[:, q * CENTROIDS:(q + 1) * CENTROIDS]
            m = jnp.max(seg, axis=1, keepdims=True)
            cand = jnp.where(seg == m, iota, float(CENTROIDS))
            idx = jnp.min(cand, axis=1, keepdims=True)
            hots.append((iota == idx).astype(jnp.bfloat16))
    hot = jnp.concatenate(hots, axis=1)
    out_ref[...] = jax.lax.dot_general(
        hot, c_ref[...], (((1,), (0,)), ((), ())),
        preferred_element_type=jnp.float32)


@jax.jit
def kernel(vecs, codebook):
    cbt = codebook.transpose(0, 2, 1)
    return pl.pallas_call(
        _quant_group,
        grid=(NGROUPS,),
        in_specs=[
            pl.BlockSpec((BATCH, GROUP * DSUB), lambda g: (0, g)),
            pl.BlockSpec((GROUP, DSUB, CENTROIDS), lambda g: (g, 0, 0)),
            pl.BlockSpec((GROUP, CENTROIDS, DSUB), lambda g: (g, 0, 0)),
        ],
        out_specs=pl.BlockSpec((BATCH, GROUP * DSUB), lambda g: (0, g)),
        out_shape=jax.ShapeDtypeStruct((BATCH, EMBED), jnp.float32),
        scratch_shapes=[
            pltpu.VMEM((GROUP * DSUB, SEG), jnp.float32),
            pltpu.VMEM((SEG, GROUP * DSUB), jnp.bfloat16),
        ],
    )(vecs, cbt, codebook)


# speedup vs baseline: 2.3665x; 2.0251x over previous
import jax
import jax.numpy as jnp
from jax.experimental import pallas as pl
from jax.experimental.pallas import tpu as pltpu

BATCH = 1024
EMBED = 768
PARTITION = 96
CENTROIDS = 256
DSUB = 8
GROUP = 16
NGROUPS = PARTITION // GROUP
SEG = GROUP * CENTROIDS
GW = GROUP * DSUB


def _quant_group(vec_ref, cbt_ref, cb_ref, out_ref, w_ref, c_ref):
    w_ref[...] = jnp.zeros((3 * GW, SEG), jnp.bfloat16)
    c_ref[...] = jnp.zeros((SEG, GW), jnp.bfloat16)
    cns = []
    for q in range(GROUP):
        cq = cbt_ref[q]
        ch = cq.astype(jnp.bfloat16)
        cl = (cq - ch.astype(jnp.float32)).astype(jnp.bfloat16)
        rows = slice(q * DSUB, (q + 1) * DSUB)
        cols = slice(q * CENTROIDS, (q + 1) * CENTROIDS)
        w_ref[rows, cols] = 2.0 * ch
        w_ref[GW + q * DSUB:GW + (q + 1) * DSUB, cols] = 2.0 * cl
        w_ref[2 * GW + q * DSUB:2 * GW + (q + 1) * DSUB, cols] = 2.0 * ch
        c_ref[cols, rows] = cb_ref[q].astype(jnp.bfloat16)
        cns.append(jnp.sum(cq * cq, axis=0, keepdims=True))
    cnorm = jnp.concatenate(cns, axis=1)

    v = vec_ref[...]
    vh = v.astype(jnp.bfloat16)
    vl = (v - vh.astype(jnp.float32)).astype(jnp.bfloat16)
    v3 = jnp.concatenate([vh, vh, vl], axis=1)
    scores = jax.lax.dot_general(
        v3, w_ref[...], (((1,), (0,)), ((), ())),
        preferred_element_type=jnp.float32)
    adj = scores - cnorm

    iota = jax.lax.broadcasted_iota(
        jnp.int32, (BATCH, CENTROIDS), 1).astype(jnp.float32)
    hots = []
    for q in range(GROUP):
        seg = adj[:, q * CENTROIDS:(q + 1) * CENTROIDS]
        m = jnp.max(seg, axis=1, keepdims=True)
        cand = jnp.where(seg == m, iota, float(CENTROIDS))
        idx = jnp.min(cand, axis=1, keepdims=True)
        hots.append((iota == idx).astype(jnp.bfloat16))
    hot = jnp.concatenate(hots, axis=1)
    out_ref[...] = jax.lax.dot_general(
        hot, c_ref[...], (((1,), (0,)), ((), ())),
        preferred_element_type=jnp.float32)


@jax.jit
def kernel(vecs, codebook):
    cbt = codebook.transpose(0, 2, 1)
    return pl.pallas_call(
        _quant_group,
        grid=(NGROUPS,),
        in_specs=[
            pl.BlockSpec((BATCH, GW), lambda g: (0, g)),
            pl.BlockSpec((GROUP, DSUB, CENTROIDS), lambda g: (g, 0, 0)),
            pl.BlockSpec((GROUP, CENTROIDS, DSUB), lambda g: (g, 0, 0)),
        ],
        out_specs=pl.BlockSpec((BATCH, GW), lambda g: (0, g)),
        out_shape=jax.ShapeDtypeStruct((BATCH, EMBED), jnp.float32),
        scratch_shapes=[
            pltpu.VMEM((3 * GW, SEG), jnp.bfloat16),
            pltpu.VMEM((SEG, GW), jnp.bfloat16),
        ],
    )(vecs, cbt, codebook)
